# baseline (speedup 1.0000x reference)
import jax
import jax.numpy as jnp
from jax import lax
from jax.experimental import pallas as pl
from jax.experimental.pallas import tpu as pltpu
from jax.experimental.pallas import tpu_sc as plsc

VOCAB = 100000
DIM = 128
B = 4096
L = 50
EPS = 1e-05

NC = 2
NS = 16
NW = NC * NS
N = B * L
PER_W = N // NW
CB = 100
NCHUNK = PER_W // CB
ND = DIM // 16
SEQ_PER_CB = CB // L


def _body(idx_hbm, table_hbm, pos_hbm, scale_hbm, bias_hbm, out_hbm,
          idx_v, buf0, buf1, pos_v, sb_v, gsem, osem):
    c = lax.axis_index("c")
    s = lax.axis_index("s")
    wid = s * NC + c

    pltpu.async_copy(pos_hbm, pos_v, gsem)
    pltpu.async_copy(scale_hbm, sb_v.at[0], gsem)
    pltpu.async_copy(bias_hbm, sb_v.at[1], gsem)
    pltpu.async_copy(idx_hbm.at[wid], idx_v, gsem)
    pltpu.make_async_copy(pos_hbm, pos_v, gsem).wait()
    pltpu.make_async_copy(scale_hbm, sb_v.at[0], gsem).wait()
    pltpu.make_async_copy(bias_hbm, sb_v.at[1], gsem).wait()
    pltpu.make_async_copy(idx_hbm.at[wid], idx_v, gsem).wait()

    scs = [sb_v[0, pl.ds(d * 16, 16)] for d in range(ND)]
    bis = [sb_v[1, pl.ds(d * 16, 16)] for d in range(ND)]

    inv_dim = 1.0 / DIM
    lanes = lax.iota(jnp.int32, 16)
    gd = lax.GatherDimensionNumbers(
        offset_dims=(), collapsed_slice_dims=(0,), start_index_map=(0,))

    def lane_sum(v):
        for sh in (8, 4, 2, 1):
            perm = lanes ^ sh
            v = v + lax.gather(v, perm[:, None], gd, (1,),
                               mode=lax.GatherScatterMode.PROMISE_IN_BOUNDS)
        return v

    def make_row_body(rows_v, base):
        def row_body(p):
            r = base + p
            vs = []
            ssum = None
            qsum = None
            for d in range(ND):
                x = rows_v[r, pl.ds(d * 16, 16)] + pos_v[p, pl.ds(d * 16, 16)]
                vs.append(x)
                ssum = x if ssum is None else ssum + x
                qsum = x * x if qsum is None else qsum + x * x
            mean_v = lane_sum(ssum) * inv_dim
            q_v = lane_sum(qsum) * inv_dim
            var_v = q_v - mean_v * mean_v + EPS
            i = lax.bitcast_convert_type(var_v, jnp.int32)
            i = 0x5F3759DF - (i >> 1)
            y = lax.bitcast_convert_type(i, jnp.float32)
            for _ in range(2):
                y = y * (1.5 - 0.5 * var_v * y * y)
            for d in range(ND):
                rows_v[r, pl.ds(d * 16, 16)] = (vs[d] - mean_v) * y * scs[d] + bis[d]
        return row_body

    bufs = (buf0, buf1)
    pltpu.async_copy(table_hbm.at[idx_v.at[0]], buf0, gsem)

    def pair_body(c2, carry):
        for b in range(2):
            ci = 2 * c2 + b
            buf = bufs[b]
            other = bufs[1 - b]
            pltpu.make_async_copy(table_hbm.at[idx_v.at[ci]], buf, gsem).wait()
            pseq = (wid * NCHUNK + ci - 1) * SEQ_PER_CB
            if b == 1:
                for k in range(SEQ_PER_CB):
                    pltpu.make_async_copy(
                        other.at[pl.ds(k * L, L)], out_hbm.at[pseq + k],
                        osem).wait()
            else:
                @pl.when(c2 > 0)
                def _():
                    for k in range(SEQ_PER_CB):
                        pltpu.make_async_copy(
                            other.at[pl.ds(k * L, L)], out_hbm.at[pseq + k],
                            osem).wait()
            cn = (ci + 1) & (NCHUNK - 1)
            pltpu.async_copy(table_hbm.at[idx_v.at[cn]], other, gsem)
            for k in range(SEQ_PER_CB):
                plsc.parallel_loop(0, L, 1, unroll=10)(
                    make_row_body(buf, k * L))
            seq = (wid * NCHUNK + ci) * SEQ_PER_CB
            for k in range(SEQ_PER_CB):
                pltpu.async_copy(buf.at[pl.ds(k * L, L)], out_hbm.at[seq + k],
                                 osem)
        return carry

    lax.fori_loop(0, NCHUNK // 2, pair_body, 0, unroll=False)
    pltpu.make_async_copy(table_hbm.at[idx_v.at[0]], buf0, gsem).wait()
    lseq = (wid * NCHUNK + NCHUNK - 1) * SEQ_PER_CB
    for k in range(SEQ_PER_CB):
        pltpu.make_async_copy(buf1.at[pl.ds(k * L, L)], out_hbm.at[lseq + k],
                              osem).wait()


@jax.jit
def _run(idx, table, pos, ln_scale, ln_bias):
    mesh = plsc.VectorSubcoreMesh(
        core_axis_name="c", subcore_axis_name="s",
        num_cores=NC, num_subcores=NS)
    f = pl.kernel(
        _body,
        out_type=jax.ShapeDtypeStruct((B, L, DIM), jnp.float32),
        mesh=mesh,
        scratch_types=[
            pltpu.VMEM((NCHUNK, CB), jnp.int32),
            pltpu.VMEM((CB, DIM), jnp.float32),
            pltpu.VMEM((CB, DIM), jnp.float32),
            pltpu.VMEM((L, DIM), jnp.float32),
            pltpu.VMEM((2, DIM), jnp.float32),
            pltpu.SemaphoreType.DMA,
            pltpu.SemaphoreType.DMA,
        ],
    )
    return f(idx, table, pos, ln_scale, ln_bias)


def kernel(inputs, table, pos_emb, ln_scale, ln_bias):
    idx = inputs.astype(jnp.int32).reshape(NW, NCHUNK, CB)
    pos = pos_emb[:L]
    return _run(idx, table, pos, ln_scale, ln_bias)


# speedup vs baseline: 1.1813x; 1.1813x over previous
import jax
import jax.numpy as jnp
from jax import lax
from jax.experimental import pallas as pl
from jax.experimental.pallas import tpu as pltpu
from jax.experimental.pallas import tpu_sc as plsc

VOCAB = 100000
DIM = 128
B = 4096
L = 50
EPS = 1e-05

NC = 2
NS = 16
NW = NC * NS
N = B * L
PER_W = N // NW
CB = 200
GB = 100
NG = CB // GB
NCHUNK = PER_W // CB
ND = DIM // 16
SEQ_PER_CB = CB // L


def _body(idx_hbm, table_hbm, pos_hbm, scale_hbm, bias_hbm, out_hbm,
          idx_v, buf0, buf1, pos_v, sb_v, gsem, osem):
    c = lax.axis_index("c")
    s = lax.axis_index("s")
    wid = s * NC + c

    pltpu.async_copy(pos_hbm, pos_v, gsem)
    pltpu.async_copy(scale_hbm, sb_v.at[0], gsem)
    pltpu.async_copy(bias_hbm, sb_v.at[1], gsem)
    pltpu.async_copy(idx_hbm.at[wid], idx_v, gsem)
    pltpu.make_async_copy(pos_hbm, pos_v, gsem).wait()
    pltpu.make_async_copy(scale_hbm, sb_v.at[0], gsem).wait()
    pltpu.make_async_copy(bias_hbm, sb_v.at[1], gsem).wait()
    pltpu.make_async_copy(idx_hbm.at[wid], idx_v, gsem).wait()

    scs = [sb_v[0, pl.ds(d * 16, 16)] for d in range(ND)]
    bis = [sb_v[1, pl.ds(d * 16, 16)] for d in range(ND)]

    inv_dim = 1.0 / DIM
    lanes = lax.iota(jnp.int32, 16)
    gd = lax.GatherDimensionNumbers(
        offset_dims=(), collapsed_slice_dims=(0,), start_index_map=(0,))

    def lane_sum(v):
        for sh in (8, 4, 2, 1):
            perm = lanes ^ sh
            v = v + lax.gather(v, perm[:, None], gd, (1,),
                               mode=lax.GatherScatterMode.PROMISE_IN_BOUNDS)
        return v

    def make_row_body(rows_v, base):
        def row_body(p):
            r = base + p
            vs = []
            ssum = None
            qsum = None
            for d in range(ND):
                x = rows_v[r, pl.ds(d * 16, 16)] + pos_v[p, pl.ds(d * 16, 16)]
                vs.append(x)
                ssum = x if ssum is None else ssum + x
                qsum = x * x if qsum is None else qsum + x * x
            mean_v = lane_sum(ssum) * inv_dim
            q_v = lane_sum(qsum) * inv_dim
            var_v = q_v - mean_v * mean_v + EPS
            i = lax.bitcast_convert_type(var_v, jnp.int32)
            i = 0x5F3759DF - (i >> 1)
            y = lax.bitcast_convert_type(i, jnp.float32)
            for _ in range(2):
                y = y * (1.5 - 0.5 * var_v * y * y)
            for d in range(ND):
                rows_v[r, pl.ds(d * 16, 16)] = (vs[d] - mean_v) * y * scs[d] + bis[d]
        return row_body

    def start_gather(ci, buf):
        for j in range(NG):
            pltpu.async_copy(table_hbm.at[idx_v.at[ci, j]],
                             buf.at[pl.ds(j * GB, GB)], gsem)

    def wait_gather(ci, buf):
        for j in range(NG):
            pltpu.make_async_copy(table_hbm.at[idx_v.at[ci, j]],
                                  buf.at[pl.ds(j * GB, GB)], gsem).wait()

    bufs = (buf0, buf1)
    start_gather(0, buf0)

    def pair_body(c2, carry):
        for b in range(2):
            ci = 2 * c2 + b
            buf = bufs[b]
            other = bufs[1 - b]
            wait_gather(ci, buf)
            pseq = (wid * NCHUNK + ci - 1) * SEQ_PER_CB
            if b == 1:
                for k in range(SEQ_PER_CB):
                    pltpu.make_async_copy(
                        other.at[pl.ds(k * L, L)], out_hbm.at[pseq + k],
                        osem).wait()
            else:
                @pl.when(c2 > 0)
                def _():
                    for k in range(SEQ_PER_CB):
                        pltpu.make_async_copy(
                            other.at[pl.ds(k * L, L)], out_hbm.at[pseq + k],
                            osem).wait()
            cn = (ci + 1) & (NCHUNK - 1)
            start_gather(cn, other)
            for k in range(SEQ_PER_CB):
                plsc.parallel_loop(0, L, 1, unroll=5)(
                    make_row_body(buf, k * L))
            seq = (wid * NCHUNK + ci) * SEQ_PER_CB
            for k in range(SEQ_PER_CB):
                pltpu.async_copy(buf.at[pl.ds(k * L, L)], out_hbm.at[seq + k],
                                 osem)
        return carry

    lax.fori_loop(0, NCHUNK // 2, pair_body, 0, unroll=False)
    wait_gather(0, buf0)
    lseq = (wid * NCHUNK + NCHUNK - 1) * SEQ_PER_CB
    for k in range(SEQ_PER_CB):
        pltpu.make_async_copy(buf1.at[pl.ds(k * L, L)], out_hbm.at[lseq + k],
                              osem).wait()


@jax.jit
def _run(idx, table, pos, ln_scale, ln_bias):
    mesh = plsc.VectorSubcoreMesh(
        core_axis_name="c", subcore_axis_name="s",
        num_cores=NC, num_subcores=NS)
    f = pl.kernel(
        _body,
        out_type=jax.ShapeDtypeStruct((B, L, DIM), jnp.float32),
        mesh=mesh,
        scratch_types=[
            pltpu.VMEM((NCHUNK, NG, GB), jnp.int32),
            pltpu.VMEM((CB, DIM), jnp.float32),
            pltpu.VMEM((CB, DIM), jnp.float32),
            pltpu.VMEM((L, DIM), jnp.float32),
            pltpu.VMEM((2, DIM), jnp.float32),
            pltpu.SemaphoreType.DMA,
            pltpu.SemaphoreType.DMA,
        ],
    )
    return f(idx, table, pos, ln_scale, ln_bias)


def kernel(inputs, table, pos_emb, ln_scale, ln_bias):
    idx = inputs.astype(jnp.int32).reshape(NW, NCHUNK, NG, GB)
    pos = pos_emb[:L]
    return _run(idx, table, pos, ln_scale, ln_bias)


# speedup vs baseline: 1.1865x; 1.0044x over previous
import jax
import jax.numpy as jnp
from jax import lax
from jax.experimental import pallas as pl
from jax.experimental.pallas import tpu as pltpu
from jax.experimental.pallas import tpu_sc as plsc

VOCAB = 100000
DIM = 128
B = 4096
L = 50
EPS = 1e-05

NC = 2
NS = 16
NW = NC * NS
N = B * L
PER_W = N // NW
CB = 100
GB = 100
NG = CB // GB
NCHUNK = PER_W // CB
ND = DIM // 16
SEQ_PER_CB = CB // L


def _body(idx_hbm, table_hbm, pos_hbm, scale_hbm, bias_hbm, out_hbm,
          idx_v, buf0, buf1, pos_v, sb_v, gsem, osem):
    c = lax.axis_index("c")
    s = lax.axis_index("s")
    wid = s * NC + c

    pltpu.async_copy(pos_hbm, pos_v, gsem)
    pltpu.async_copy(scale_hbm, sb_v.at[0], gsem)
    pltpu.async_copy(bias_hbm, sb_v.at[1], gsem)
    pltpu.async_copy(idx_hbm.at[wid], idx_v, gsem)
    pltpu.make_async_copy(pos_hbm, pos_v, gsem).wait()
    pltpu.make_async_copy(scale_hbm, sb_v.at[0], gsem).wait()
    pltpu.make_async_copy(bias_hbm, sb_v.at[1], gsem).wait()
    pltpu.make_async_copy(idx_hbm.at[wid], idx_v, gsem).wait()

    scs = [sb_v[0, pl.ds(d * 16, 16)] for d in range(ND)]
    bis = [sb_v[1, pl.ds(d * 16, 16)] for d in range(ND)]

    inv_dim = 1.0 / DIM
    lanes = lax.iota(jnp.int32, 16)
    gd = lax.GatherDimensionNumbers(
        offset_dims=(), collapsed_slice_dims=(0,), start_index_map=(0,))

    def lane_sum(v):
        for sh in (8, 4, 2, 1):
            perm = lanes ^ sh
            v = v + lax.gather(v, perm[:, None], gd, (1,),
                               mode=lax.GatherScatterMode.PROMISE_IN_BOUNDS)
        return v

    def make_row_body(rows_v, base):
        def row_body(p):
            r = base + p
            vs = []
            ssum = None
            qsum = None
            for d in range(ND):
                x = rows_v[r, pl.ds(d * 16, 16)] + pos_v[p, pl.ds(d * 16, 16)]
                vs.append(x)
                ssum = x if ssum is None else ssum + x
                qsum = x * x if qsum is None else qsum + x * x
            mean_v = lane_sum(ssum) * inv_dim
            q_v = lane_sum(qsum) * inv_dim
            var_v = q_v - mean_v * mean_v + EPS
            i = lax.bitcast_convert_type(var_v, jnp.int32)
            i = 0x5F3759DF - (i >> 1)
            y = lax.bitcast_convert_type(i, jnp.float32)
            for _ in range(2):
                y = y * (1.5 - 0.5 * var_v * y * y)
            for d in range(ND):
                rows_v[r, pl.ds(d * 16, 16)] = (vs[d] - mean_v) * y * scs[d] + bis[d]
        return row_body

    def start_gather(ci, buf):
        for j in range(NG):
            pltpu.async_copy(table_hbm.at[idx_v.at[ci, j]],
                             buf.at[pl.ds(j * GB, GB)], gsem)

    def wait_gather(ci, buf):
        for j in range(NG):
            pltpu.make_async_copy(table_hbm.at[idx_v.at[ci, j]],
                                  buf.at[pl.ds(j * GB, GB)], gsem).wait()

    bufs = (buf0, buf1)
    start_gather(0, buf0)

    def pair_body(c2, carry):
        for b in range(2):
            ci = 2 * c2 + b
            buf = bufs[b]
            other = bufs[1 - b]
            wait_gather(ci, buf)
            pseq = (wid * NCHUNK + ci - 1) * SEQ_PER_CB
            if b == 1:
                for k in range(SEQ_PER_CB):
                    pltpu.make_async_copy(
                        other.at[pl.ds(k * L, L)], out_hbm.at[pseq + k],
                        osem).wait()
            else:
                @pl.when(c2 > 0)
                def _():
                    for k in range(SEQ_PER_CB):
                        pltpu.make_async_copy(
                            other.at[pl.ds(k * L, L)], out_hbm.at[pseq + k],
                            osem).wait()
            cn = (ci + 1) & (NCHUNK - 1)
            start_gather(cn, other)
            for k in range(SEQ_PER_CB):
                plsc.parallel_loop(0, L, 1, unroll=3)(
                    make_row_body(buf, k * L))
            seq = (wid * NCHUNK + ci) * SEQ_PER_CB
            for k in range(SEQ_PER_CB):
                pltpu.async_copy(buf.at[pl.ds(k * L, L)], out_hbm.at[seq + k],
                                 osem)
        return carry

    lax.fori_loop(0, NCHUNK // 2, pair_body, 0, unroll=False)
    wait_gather(0, buf0)
    lseq = (wid * NCHUNK + NCHUNK - 1) * SEQ_PER_CB
    for k in range(SEQ_PER_CB):
        pltpu.make_async_copy(buf1.at[pl.ds(k * L, L)], out_hbm.at[lseq + k],
                              osem).wait()


@jax.jit
def _run(idx, table, pos, ln_scale, ln_bias):
    mesh = plsc.VectorSubcoreMesh(
        core_axis_name="c", subcore_axis_name="s",
        num_cores=NC, num_subcores=NS)
    f = pl.kernel(
        _body,
        out_type=jax.ShapeDtypeStruct((B, L, DIM), jnp.float32),
        mesh=mesh,
        scratch_types=[
            pltpu.VMEM((NCHUNK, NG, GB), jnp.int32),
            pltpu.VMEM((CB, DIM), jnp.float32),
            pltpu.VMEM((CB, DIM), jnp.float32),
            pltpu.VMEM((L, DIM), jnp.float32),
            pltpu.VMEM((2, DIM), jnp.float32),
            pltpu.SemaphoreType.DMA,
            pltpu.SemaphoreType.DMA,
        ],
    )
    return f(idx, table, pos, ln_scale, ln_bias)


def kernel(inputs, table, pos_emb, ln_scale, ln_bias):
    idx = inputs.astype(jnp.int32).reshape(NW, NCHUNK, NG, GB)
    pos = pos_emb[:L]
    return _run(idx, table, pos, ln_scale, ln_bias)


# speedup vs baseline: 1.2493x; 1.0529x over previous
import jax
import jax.numpy as jnp
from jax import lax
from jax.experimental import pallas as pl
from jax.experimental.pallas import tpu as pltpu
from jax.experimental.pallas import tpu_sc as plsc

VOCAB = 100000
DIM = 128
B = 4096
L = 50
EPS = 1e-05

NC = 2
NS = 16
NW = NC * NS
N = B * L
PER_W = N // NW
CB = 100
GB = 100
NG = CB // GB
NCHUNK = PER_W // CB
ND = DIM // 16
SEQ_PER_CB = CB // L


def _body(idx_hbm, table_hbm, pos_hbm, scale_hbm, bias_hbm, out_hbm,
          idx_v, buf0, buf1, pos_v, sb_v, gsem, osem):
    c = lax.axis_index("c")
    s = lax.axis_index("s")
    wid = s * NC + c

    pltpu.async_copy(pos_hbm, pos_v, gsem)
    pltpu.async_copy(scale_hbm, sb_v.at[0], gsem)
    pltpu.async_copy(bias_hbm, sb_v.at[1], gsem)
    pltpu.async_copy(idx_hbm.at[wid], idx_v, gsem)
    pltpu.make_async_copy(pos_hbm, pos_v, gsem).wait()
    pltpu.make_async_copy(scale_hbm, sb_v.at[0], gsem).wait()
    pltpu.make_async_copy(bias_hbm, sb_v.at[1], gsem).wait()
    pltpu.make_async_copy(idx_hbm.at[wid], idx_v, gsem).wait()

    scs = [sb_v[0, pl.ds(d * 16, 16)] for d in range(ND)]
    bis = [sb_v[1, pl.ds(d * 16, 16)] for d in range(ND)]

    inv_dim = 1.0 / DIM
    lanes = lax.iota(jnp.int32, 16)
    gd = lax.GatherDimensionNumbers(
        offset_dims=(), collapsed_slice_dims=(0,), start_index_map=(0,))

    def lane_sum(v):
        for sh in (8, 4, 2, 1):
            perm = lanes ^ sh
            v = v + lax.gather(v, perm[:, None], gd, (1,),
                               mode=lax.GatherScatterMode.PROMISE_IN_BOUNDS)
        return v

    def make_row_body(rows_v, base):
        def row_body(p):
            r = base + p
            vs = []
            ssum = None
            qsum = None
            for d in range(ND):
                x = rows_v[r, pl.ds(d * 16, 16)] + pos_v[p, pl.ds(d * 16, 16)]
                vs.append(x)
                ssum = x if ssum is None else ssum + x
                qsum = x * x if qsum is None else qsum + x * x
            mean_v = lane_sum(ssum) * inv_dim
            q_v = lane_sum(qsum) * inv_dim
            var_v = q_v - mean_v * mean_v + EPS
            i = lax.bitcast_convert_type(var_v, jnp.int32)
            i = 0x5F3759DF - (i >> 1)
            y = lax.bitcast_convert_type(i, jnp.float32)
            for _ in range(1):
                y = y * (1.5 - 0.5 * var_v * y * y)
            for d in range(ND):
                rows_v[r, pl.ds(d * 16, 16)] = (vs[d] - mean_v) * y * scs[d] + bis[d]
        return row_body

    def start_gather(ci, buf):
        for j in range(NG):
            pltpu.async_copy(table_hbm.at[idx_v.at[ci, j]],
                             buf.at[pl.ds(j * GB, GB)], gsem)

    def wait_gather(ci, buf):
        for j in range(NG):
            pltpu.make_async_copy(table_hbm.at[idx_v.at[ci, j]],
                                  buf.at[pl.ds(j * GB, GB)], gsem).wait()

    bufs = (buf0, buf1)
    start_gather(0, buf0)

    def pair_body(c2, carry):
        for b in range(2):
            ci = 2 * c2 + b
            buf = bufs[b]
            other = bufs[1 - b]
            wait_gather(ci, buf)
            pseq = (wid * NCHUNK + ci - 1) * SEQ_PER_CB
            if b == 1:
                for k in range(SEQ_PER_CB):
                    pltpu.make_async_copy(
                        other.at[pl.ds(k * L, L)], out_hbm.at[pseq + k],
                        osem).wait()
            else:
                @pl.when(c2 > 0)
                def _():
                    for k in range(SEQ_PER_CB):
                        pltpu.make_async_copy(
                            other.at[pl.ds(k * L, L)], out_hbm.at[pseq + k],
                            osem).wait()
            cn = (ci + 1) & (NCHUNK - 1)
            start_gather(cn, other)
            for k in range(SEQ_PER_CB):
                plsc.parallel_loop(0, L, 1, unroll=5)(
                    make_row_body(buf, k * L))
            seq = (wid * NCHUNK + ci) * SEQ_PER_CB
            for k in range(SEQ_PER_CB):
                pltpu.async_copy(buf.at[pl.ds(k * L, L)], out_hbm.at[seq + k],
                                 osem)
        return carry

    lax.fori_loop(0, NCHUNK // 2, pair_body, 0, unroll=False)
    wait_gather(0, buf0)
    lseq = (wid * NCHUNK + NCHUNK - 1) * SEQ_PER_CB
    for k in range(SEQ_PER_CB):
        pltpu.make_async_copy(buf1.at[pl.ds(k * L, L)], out_hbm.at[lseq + k],
                              osem).wait()


@jax.jit
def _run(idx, table, pos, ln_scale, ln_bias):
    mesh = plsc.VectorSubcoreMesh(
        core_axis_name="c", subcore_axis_name="s",
        num_cores=NC, num_subcores=NS)
    f = pl.kernel(
        _body,
        out_type=jax.ShapeDtypeStruct((B, L, DIM), jnp.float32),
        mesh=mesh,
        scratch_types=[
            pltpu.VMEM((NCHUNK, NG, GB), jnp.int32),
            pltpu.VMEM((CB, DIM), jnp.float32),
            pltpu.VMEM((CB, DIM), jnp.float32),
            pltpu.VMEM((L, DIM), jnp.float32),
            pltpu.VMEM((2, DIM), jnp.float32),
            pltpu.SemaphoreType.DMA,
            pltpu.SemaphoreType.DMA,
        ],
    )
    return f(idx, table, pos, ln_scale, ln_bias)


def kernel(inputs, table, pos_emb, ln_scale, ln_bias):
    idx = inputs.astype(jnp.int32).reshape(NW, NCHUNK, NG, GB)
    pos = pos_emb[:L]
    return _run(idx, table, pos, ln_scale, ln_bias)
